# Initial kernel scaffold; baseline (speedup 1.0000x reference)
import jax
import jax.numpy as jnp
from jax.experimental import pallas as pl


def kernel(x, edge_index, batch, W1, b1, W2, b2, W3, b3, bn1_g, bn1_b, bn2_g, bn2_b, bn3_g, bn3_b, fc1_W, fc1_b, fc2_W, fc2_b):
    raise NotImplementedError("write your pallas kernel here")


# speedup vs baseline: 11.0577x; 11.0577x over previous
import functools

import jax
import jax.numpy as jnp
from jax import lax
from jax.experimental import pallas as pl
from jax.experimental.pallas import tpu as pltpu
from jax.experimental.pallas import tpu_sc as plsc

N = 10000
E = 320000
D = 128
G = 64
EPS = 1e-5

NC = 2
NS = 16
NW = NC * NS
CHUNK = 128
CH = -(-E // (NW * CHUNK))
EP = NW * CH * CHUNK
NP = 10112
RPT = NP // NS

_mesh = plsc.VectorSubcoreMesh(core_axis_name="c", subcore_axis_name="s",
                               num_cores=NC, num_subcores=NS)


def _deg_body(dstp, zeros, ones, out, idx_v, ones_v, deg_sh):
    cid = lax.axis_index("c")
    sid = lax.axis_index("s")
    wid = sid * NC + cid
    pltpu.sync_copy(dstp.at[wid], idx_v)
    pltpu.sync_copy(ones, ones_v)
    pltpu.sync_copy(zeros.at[pl.ds(sid * RPT, RPT)],
                    deg_sh.at[pl.ds(sid * RPT, RPT)])
    plsc.subcore_barrier()

    @pl.loop(0, CH)
    def _chunk(j):
        pltpu.sync_copy(ones_v, deg_sh.at[idx_v.at[j]], add=True)

    plsc.subcore_barrier()
    pltpu.sync_copy(deg_sh.at[pl.ds(sid * RPT, RPT)],
                    out.at[pl.ds(cid * NP + sid * RPT, RPT)])


_deg_call = pl.kernel(
    _deg_body,
    out_type=jax.ShapeDtypeStruct((NC * NP, D), jnp.float32),
    mesh=_mesh,
    scratch_types=[
        pltpu.VMEM((CH, CHUNK), jnp.int32),
        pltpu.VMEM((CHUNK, D), jnp.float32),
        pltpu.VMEM_SHARED((NP, D), jnp.float32),
    ],
)


def _agg_body(hs, srcp, dstp, zeros, out, src_v, dst_v, gbuf, agg_sh, sem):
    cid = lax.axis_index("c")
    sid = lax.axis_index("s")
    wid = sid * NC + cid
    pltpu.sync_copy(srcp.at[wid], src_v)
    pltpu.sync_copy(dstp.at[wid], dst_v)
    pltpu.sync_copy(zeros.at[pl.ds(sid * RPT, RPT)],
                    agg_sh.at[pl.ds(sid * RPT, RPT)])
    plsc.subcore_barrier()

    @pl.loop(0, CH)
    def _chunk(j):
        pltpu.async_copy(hs.at[src_v.at[j]], gbuf, sem).wait()
        pltpu.sync_copy(gbuf, agg_sh.at[dst_v.at[j]], add=True)

    plsc.subcore_barrier()
    pltpu.sync_copy(agg_sh.at[pl.ds(sid * RPT, RPT)],
                    out.at[pl.ds(cid * NP + sid * RPT, RPT)])


_agg_call = pl.kernel(
    _agg_body,
    out_type=jax.ShapeDtypeStruct((NC * NP, D), jnp.float32),
    mesh=_mesh,
    scratch_types=[
        pltpu.VMEM((CH, CHUNK), jnp.int32),
        pltpu.VMEM((CH, CHUNK), jnp.int32),
        pltpu.VMEM((CHUNK, D), jnp.float32),
        pltpu.VMEM_SHARED((NP, D), jnp.float32),
        pltpu.SemaphoreType.DMA,
    ],
)


def _dis(degp_ref):
    deg = (degp_ref[0:N, 0:1] + degp_ref[NP:NP + N, 0:1]) + 1.0
    return lax.rsqrt(deg)


def _t1_body(x_ref, w_ref, degp_ref, hs_ref):
    hs_ref[...] = _dis(degp_ref) * jnp.dot(
        x_ref[...], w_ref[...], preferred_element_type=jnp.float32)


def _layer_epilogue(aggp_ref, hs_ref, degp_ref, b_ref, g_ref, bb_ref):
    dis = _dis(degp_ref)
    agg = aggp_ref[0:N, :] + aggp_ref[NP:NP + N, :]
    a = jax.nn.relu(dis * (agg + hs_ref[...]) + b_ref[...])
    gscale = g_ref[...] * (1.0 / jnp.sqrt(1.0 + EPS))
    return a * gscale + bb_ref[...]


def _tmid_body(aggp_ref, hs_ref, degp_ref, b_ref, g_ref, bb_ref, w_ref,
               hsn_ref):
    a = _layer_epilogue(aggp_ref, hs_ref, degp_ref, b_ref, g_ref, bb_ref)
    hsn_ref[...] = _dis(degp_ref) * jnp.dot(
        a, w_ref[...], preferred_element_type=jnp.float32)


def _t4_body(aggp_ref, hs_ref, degp_ref, b_ref, g_ref, bb_ref, batch_ref,
             fc1w_ref, fc1b_ref, fc2w_ref, fc2b_ref, out_ref):
    a3 = _layer_epilogue(aggp_ref, hs_ref, degp_ref, b_ref, g_ref, bb_ref)
    gid = lax.broadcasted_iota(jnp.int32, (N, G), 1)
    onehot = (batch_ref[...] == gid).astype(jnp.float32)
    dn = (((0,), (0,)), ((), ()))
    sums = lax.dot_general(onehot, a3, dn,
                           preferred_element_type=jnp.float32)
    cnts = lax.dot_general(onehot, jnp.ones((N, 1), jnp.float32), dn,
                           preferred_element_type=jnp.float32)
    mean = sums / jnp.maximum(cnts, 1.0)
    dn_t = (((1,), (1,)), ((), ()))
    p = jax.nn.relu(
        lax.dot_general(mean, fc1w_ref[...], dn_t,
                        preferred_element_type=jnp.float32) + fc1b_ref[...])
    out_ref[...] = lax.dot_general(
        p, fc2w_ref[...], dn_t,
        preferred_element_type=jnp.float32) + fc2b_ref[...]


def _tc(body, out_shape):
    return pl.pallas_call(body, out_shape=out_shape)


def kernel(x, edge_index, batch, W1, b1, W2, b2, W3, b3,
           bn1_g, bn1_b, bn2_g, bn2_b, bn3_g, bn3_b,
           fc1_W, fc1_b, fc2_W, fc2_b):
    src = edge_index[0]
    dst = edge_index[1]
    pad = EP - E
    srcp = jnp.concatenate(
        [src, jnp.zeros((pad,), jnp.int32)]).reshape(NW, CH, CHUNK)
    dstp = jnp.concatenate(
        [dst, jnp.full((pad,), N, jnp.int32)]).reshape(NW, CH, CHUNK)
    zeros = jnp.zeros((NP, D), jnp.float32)
    ones128 = jnp.ones((CHUNK, D), jnp.float32)
    batch2 = batch.reshape(N, 1)

    r1 = lambda v: v.reshape(1, D)

    degp = _deg_call(dstp, zeros, ones128)

    f32 = jnp.float32
    hs1 = _tc(_t1_body, jax.ShapeDtypeStruct((N, D), f32))(x, W1, degp)
    agg1 = _agg_call(hs1, srcp, dstp, zeros)
    hs2 = _tc(_tmid_body, jax.ShapeDtypeStruct((N, D), f32))(
        agg1, hs1, degp, r1(b1), r1(bn1_g), r1(bn1_b), W2)
    agg2 = _agg_call(hs2, srcp, dstp, zeros)
    hs3 = _tc(_tmid_body, jax.ShapeDtypeStruct((N, D), f32))(
        agg2, hs2, degp, r1(b2), r1(bn2_g), r1(bn2_b), W3)
    agg3 = _agg_call(hs3, srcp, dstp, zeros)
    out = _tc(_t4_body, jax.ShapeDtypeStruct((G, D), f32))(
        agg3, hs3, degp, r1(b3), r1(bn3_g), r1(bn3_b), batch2,
        fc1_W, r1(fc1_b), fc2_W, r1(fc2_b))
    return out


# speedup vs baseline: 25.8623x; 2.3388x over previous
import jax
import jax.numpy as jnp
from jax import lax
from jax.experimental import pallas as pl
from jax.experimental.pallas import tpu as pltpu
from jax.experimental.pallas import tpu_sc as plsc

N = 10000
E = 320000
D = 128
G = 64
EPS = 1e-5

NC = 2
NS = 16
CHUNK = 128
BL = 4
BPT = 20
TOTB = NC * NS * BPT
EP = TOTB * BL * CHUNK
CHB = BPT
NP = 10112
RPT = NP // NS

_mesh = plsc.VectorSubcoreMesh(core_axis_name="c", subcore_axis_name="s",
                               num_cores=NC, num_subcores=NS)


def _fill(buf, value):
    val = jnp.full((16,), value, jnp.float32)

    @pl.loop(0, CHUNK)
    def _row(i):
        for j in range(D // 16):
            buf[i, pl.ds(j * 16, 16)] = val


def _zero_rows(zbuf, shared, sid):
    full, rem = RPT // CHUNK, RPT % CHUNK
    for t in range(full):
        pltpu.sync_copy(zbuf, shared.at[pl.ds(sid * RPT + t * CHUNK, CHUNK)])
    if rem:
        pltpu.sync_copy(zbuf.at[pl.ds(0, rem)],
                        shared.at[pl.ds(sid * RPT + full * CHUNK, rem)])


def _deg_body(sd, out, idx_v, ones_v, deg_sh):
    cid = lax.axis_index("c")
    sid = lax.axis_index("s")
    base = cid * NS * CHB + sid * CHB
    _fill(ones_v, 0.0)
    _zero_rows(ones_v, deg_sh, sid)
    _fill(ones_v, 1.0)
    plsc.subcore_barrier()

    @pl.loop(0, CHB)
    def _block(b):
        pltpu.sync_copy(sd.at[base + b], idx_v)
        for r in range(BL):
            pltpu.sync_copy(ones_v, deg_sh.at[idx_v.at[BL + r]], add=True)

    plsc.subcore_barrier()
    pltpu.sync_copy(deg_sh.at[pl.ds(sid * RPT, RPT)],
                    out.at[pl.ds(cid * NP + sid * RPT, RPT)])


_deg_call = pl.kernel(
    _deg_body,
    out_type=jax.ShapeDtypeStruct((NC * NP, D), jnp.float32),
    mesh=_mesh,
    scratch_types=[
        pltpu.VMEM((2 * BL, CHUNK), jnp.int32),
        pltpu.VMEM((CHUNK, D), jnp.float32),
        pltpu.VMEM_SHARED((NP, D), jnp.float32),
    ],
)


def _agg_body(hs, sd, out, i0, i1, g0, g1, agg_sh,
              semi0, semi1, semg0, semg1):
    cid = lax.axis_index("c")
    sid = lax.axis_index("s")
    base = (cid * NS + sid) * BPT
    _fill(g0, 0.0)
    _zero_rows(g0, agg_sh, sid)
    plsc.subcore_barrier()

    iblk = (i0, i1)
    isem = (semi0, semi1)
    gb = (g0, g1)
    gsem = (semg0, semg1)

    def idx_start(b, q):
        pltpu.async_copy(sd.at[base + b], iblk[q], isem[q])

    def idx_wait(b, q):
        pltpu.make_async_copy(sd.at[base + b], iblk[q], isem[q]).wait()

    def gather_start(q, r, p):
        pltpu.async_copy(hs.at[iblk[q].at[r]], gb[p], gsem[p])

    def gather_wait(q, r, p):
        pltpu.make_async_copy(hs.at[iblk[q].at[r]], gb[p], gsem[p]).wait()

    def scat(q, r, p):
        pltpu.sync_copy(gb[p], agg_sh.at[iblk[q].at[BL + r]], add=True)

    def pair(b, prefetch, last):
        for k in range(2 * BL):
            q = (k // BL) & 1
            r = k % BL
            p = k & 1
            if prefetch and k == BL:
                idx_start(b + 2, 0)
            if k < 2 * BL - 1 or not last:
                nk = k + 1
                nq, nr, npp = (nk // BL) & 1, nk % BL, nk & 1
                if nk == 2 * BL:
                    idx_wait(b + 2, 0)
                    nq, nr = 0, 0
                elif nk == BL:
                    idx_wait(b + 1, 1)
                gather_start(nq, nr, npp)
            gather_wait(q, r, p)
            scat(q, r, p)
            if prefetch and k == 2 * BL - 1:
                idx_start(b + 3, 1)

    idx_start(0, 0)
    idx_start(1, 1)
    idx_wait(0, 0)
    gather_start(0, 0, 0)

    @pl.loop(0, BPT - 2, step=2)
    def _body(b):
        pair(b, prefetch=True, last=False)

    pair(BPT - 2, prefetch=False, last=True)

    plsc.subcore_barrier()
    pltpu.sync_copy(agg_sh.at[pl.ds(sid * RPT, RPT)],
                    out.at[pl.ds(cid * NP + sid * RPT, RPT)])


_agg_call = pl.kernel(
    _agg_body,
    out_type=jax.ShapeDtypeStruct((NC * NP, D), jnp.float32),
    mesh=_mesh,
    scratch_types=[
        pltpu.VMEM((2 * BL, CHUNK), jnp.int32),
        pltpu.VMEM((2 * BL, CHUNK), jnp.int32),
        pltpu.VMEM((CHUNK, D), jnp.float32),
        pltpu.VMEM((CHUNK, D), jnp.float32),
        pltpu.VMEM_SHARED((NP, D), jnp.float32),
        pltpu.SemaphoreType.DMA,
        pltpu.SemaphoreType.DMA,
        pltpu.SemaphoreType.DMA,
        pltpu.SemaphoreType.DMA,
    ],
)


def _dis(degp_ref):
    deg = (degp_ref[0:N, 0:1] + degp_ref[NP:NP + N, 0:1]) + 1.0
    return lax.rsqrt(deg)


def _t1_body(x_ref, w_ref, degp_ref, hs_ref):
    hs_ref[...] = _dis(degp_ref) * jnp.dot(
        x_ref[...], w_ref[...], preferred_element_type=jnp.float32)


def _layer_epilogue(aggp_ref, hs_ref, degp_ref, b_ref, g_ref, bb_ref):
    dis = _dis(degp_ref)
    agg = aggp_ref[0:N, :] + aggp_ref[NP:NP + N, :]
    a = jax.nn.relu(dis * (agg + hs_ref[...]) + b_ref[...])
    gscale = g_ref[...] * (1.0 / jnp.sqrt(1.0 + EPS))
    return a * gscale + bb_ref[...]


def _tmid_body(aggp_ref, hs_ref, degp_ref, b_ref, g_ref, bb_ref, w_ref,
               hsn_ref):
    a = _layer_epilogue(aggp_ref, hs_ref, degp_ref, b_ref, g_ref, bb_ref)
    hsn_ref[...] = _dis(degp_ref) * jnp.dot(
        a, w_ref[...], preferred_element_type=jnp.float32)


def _t4_body(aggp_ref, hs_ref, degp_ref, b_ref, g_ref, bb_ref, batch_ref,
             fc1w_ref, fc1b_ref, fc2w_ref, fc2b_ref, out_ref):
    a3 = _layer_epilogue(aggp_ref, hs_ref, degp_ref, b_ref, g_ref, bb_ref)
    gid = lax.broadcasted_iota(jnp.int32, (N, G), 1)
    onehot = (batch_ref[...] == gid).astype(jnp.float32)
    dn = (((0,), (0,)), ((), ()))
    sums = lax.dot_general(onehot, a3, dn,
                           preferred_element_type=jnp.float32)
    cnts = lax.dot_general(onehot, jnp.ones((N, 1), jnp.float32), dn,
                           preferred_element_type=jnp.float32)
    mean = sums / jnp.maximum(cnts, 1.0)
    dn_t = (((1,), (1,)), ((), ()))
    p = jax.nn.relu(
        lax.dot_general(mean, fc1w_ref[...], dn_t,
                        preferred_element_type=jnp.float32) + fc1b_ref[...])
    out_ref[...] = lax.dot_general(
        p, fc2w_ref[...], dn_t,
        preferred_element_type=jnp.float32) + fc2b_ref[...]


def _tc(body, out_shape):
    return pl.pallas_call(body, out_shape=out_shape)


def kernel(x, edge_index, batch, W1, b1, W2, b2, W3, b3,
           bn1_g, bn1_b, bn2_g, bn2_b, bn3_g, bn3_b,
           fc1_W, fc1_b, fc2_W, fc2_b):
    src = edge_index[0]
    dst = edge_index[1]
    pad = EP - E
    junk_src = jnp.arange(pad, dtype=jnp.int32) % N
    srcp = jnp.concatenate([src, junk_src]).reshape(TOTB, BL, CHUNK)
    junk = N + (jnp.arange(pad, dtype=jnp.int32) % (NP - N))
    dstp = jnp.concatenate([dst, junk]).reshape(TOTB, BL, CHUNK)
    sd = jnp.concatenate([srcp, dstp], axis=1)
    batch2 = batch.reshape(N, 1)

    r1 = lambda v: v.reshape(1, D)

    degp = _deg_call(sd)

    f32 = jnp.float32
    hs1 = _tc(_t1_body, jax.ShapeDtypeStruct((N, D), f32))(x, W1, degp)
    agg1 = _agg_call(hs1, sd)
    hs2 = _tc(_tmid_body, jax.ShapeDtypeStruct((N, D), f32))(
        agg1, hs1, degp, r1(b1), r1(bn1_g), r1(bn1_b), W2)
    agg2 = _agg_call(hs2, sd)
    hs3 = _tc(_tmid_body, jax.ShapeDtypeStruct((N, D), f32))(
        agg2, hs2, degp, r1(b2), r1(bn2_g), r1(bn2_b), W3)
    agg3 = _agg_call(hs3, sd)
    out = _tc(_t4_body, jax.ShapeDtypeStruct((G, D), f32))(
        agg3, hs3, degp, r1(b3), r1(bn3_g), r1(bn3_b), batch2,
        fc1_W, r1(fc1_b), fc2_W, r1(fc2_b))
    return out
